# baseline (speedup 1.0000x reference)
import functools

import numpy as np
import jax
import jax.numpy as jnp
from jax import lax
from jax.experimental import pallas as pl
from jax.experimental.pallas import tpu as pltpu
from jax.experimental.pallas import tpu_sc as plsc

_D = 256
_ROWS = 50
_L = 16


def _build_consts(h: int, w: int):
    idx = np.zeros((2, 64), np.int32)
    wgt = np.zeros((2, 64), np.float32)
    pmap = np.zeros((2, h * w), np.int32)
    for half, n in ((0, w), (1, h)):
        coord = np.arange(n, dtype=np.float32) / np.float32(n) * np.float32(49.0)
        fi = np.floor(coord).astype(np.int32)
        ci = np.minimum(fi + 1, _ROWS - 1).astype(np.int32)
        delta = coord - np.floor(coord)
        idx[half, :n] = fi + _ROWS * half
        idx[half, 32:32 + n] = ci + _ROWS * half
        wgt[half, :n] = np.float32(1.0) - delta
        wgt[half, 32:32 + n] = delta
    p = np.arange(h * w, dtype=np.int32)
    pmap[0] = p % w
    pmap[1] = p // w
    return idx, wgt, pmap


@functools.lru_cache(maxsize=None)
def _make_sc_kernel(B: int, h: int, w: int):
    HW = h * w
    NB = _L * HW
    out_words = B * 2 * _D * HW
    mesh = plsc.VectorSubcoreMesh(core_axis_name="c", subcore_axis_name="s")

    @functools.partial(
        pl.kernel,
        mesh=mesh,
        out_type=jax.ShapeDtypeStruct((out_words,), jnp.float32),
        compiler_params=pltpu.CompilerParams(needs_layout_passes=False),
        scratch_types=[
            pltpu.VMEM((64,), jnp.int32),
            pltpu.VMEM((64,), jnp.float32),
            pltpu.VMEM((HW,), jnp.int32),
            pltpu.VMEM((_L * 2 * _ROWS,), jnp.float32),
            pltpu.VMEM((_L * 32,), jnp.float32),
            pltpu.VMEM((NB,), jnp.float32),
            pltpu.SemaphoreType.DMA,
        ],
    )
    def body(tables, idxs, wgts, pmaps, out, idx_v, wgt_v, pmap_v, tt_v,
             e_v, buf_v, sem):
        wid = lax.axis_index("s") * 2 + lax.axis_index("c")
        half = wid // 16
        grp = wid % 16
        cbase = grp * _L
        nt = 2 * _ROWS

        pltpu.sync_copy(idxs.at[half], idx_v)
        pltpu.sync_copy(wgts.at[half], wgt_v)
        pltpu.sync_copy(pmaps.at[half], pmap_v)
        pltpu.sync_copy(tables.at[pl.ds(cbase * nt, _L * nt)], tt_v)

        for ch in range(w // _L):
            fiv = idx_v[pl.ds(_L * ch, _L)]
            civ = idx_v[pl.ds(32 + _L * ch, _L)]
            wf = wgt_v[pl.ds(_L * ch, _L)]
            wc = wgt_v[pl.ds(32 + _L * ch, _L)]
            for cl in range(_L):
                vf = plsc.load_gather(tt_v, [fiv + cl * nt])
                vc = plsc.load_gather(tt_v, [civ + cl * nt])
                e_v[pl.ds(cl * 32 + _L * ch, _L)] = wf * vf + wc * vc

        def expand(k, carry):
            pm = pmap_v[pl.ds(k * _L, _L)]
            for cl in range(_L):
                v = plsc.load_gather(e_v, [pm + cl * 32])
                buf_v[pl.ds(cl * HW + k * _L, _L)] = v
            return carry

        lax.fori_loop(0, HW // _L, expand, 0)

        copies = []
        for b in range(B):
            base = (b * 2 * _D + half * _D + cbase) * HW
            copies.append(pltpu.async_copy(buf_v, out.at[pl.ds(base, NB)], sem))
        for c in copies:
            c.wait()

    return body


def kernel(x, row_embed, col_embed):
    B = x.shape[0]
    h, w = x.shape[-2], x.shape[-1]
    idx, wgt, pmap = _build_consts(h, w)
    tables = jnp.concatenate([col_embed, row_embed], axis=0).T.reshape(-1)
    out = _make_sc_kernel(B, h, w)(
        tables, jnp.asarray(idx), jnp.asarray(wgt), jnp.asarray(pmap))
    return out.reshape(B, 2 * _D, h, w)
